# baseline (speedup 1.0000x reference)
import functools
import math

import jax
import jax.numpy as jnp
from jax import lax
from jax.experimental import pallas as pl
from jax.experimental.pallas import tpu as pltpu


_S = 256


def _sort_body(a_ref, s_ref, stone_out, dist_out, ang_out):
    a = a_ref[...]
    s = s_ref[...]
    bn = a.shape[0]

    ay = a[:, 1][:, None]
    ax = a[:, 2][:, None]
    sy = s[:, 1][None, :]
    sx = s[:, 2][None, :]

    dy = sy - ay
    dx = sx - ax
    dist = jnp.sqrt(dy * dy + dx * dx)
    raw = jnp.arctan2(-dy, dx) * (180.0 / math.pi)
    ang = jnp.where(raw > 0, raw, raw + 360.0)
    stone = jnp.broadcast_to(s[:, 0][None, :], (bn, _S))

    lane = lax.broadcasted_iota(jnp.int32, (bn, _S), 1)
    idx = lane

    def stage(j, k, dist, idx, stone, ang):
        low = (lane & j) == 0
        m = low == ((lane & k) == 0)
        jm = jnp.where(low, _S - j, j)

        def other(x):
            rm = jnp.concatenate([x[:, j:], x[:, :j]], axis=1)
            rp = jnp.concatenate([x[:, -j:], x[:, :-j]], axis=1)
            return jnp.where(low, rm, rp)

        od = other(dist)
        oi = other(idx)
        ost = other(stone)
        oan = other(ang)
        lt = (dist < od) | ((dist == od) & (idx < oi))
        keep = lt == m
        return (jnp.where(keep, dist, od), jnp.where(keep, idx, oi),
                jnp.where(keep, stone, ost), jnp.where(keep, ang, oan))

    logn = 8
    for lk in range(1, logn + 1):
        k = 1 << lk
        j = k // 2
        while j >= 1:
            dist, idx, stone, ang = stage(j, k, dist, idx, stone, ang)
            j //= 2

    mask = (a[:, 0] == 0.0)[:, None]
    stone_out[...] = jnp.where(mask, stone, 0.0)
    dist_out[...] = jnp.where(mask, dist, 0.0)
    ang_out[...] = jnp.where(mask, ang, 0.0)


@jax.jit
def kernel(all_coord_input, stone_coord_input):
    a = all_coord_input.astype(jnp.float32)
    s = stone_coord_input.astype(jnp.float32)
    n = a.shape[0]
    bn = 8
    grid = n // bn

    out_shape = [jax.ShapeDtypeStruct((n, _S), jnp.float32)] * 3
    stone_s, dist_s, ang_s = pl.pallas_call(
        _sort_body,
        grid=(grid,),
        in_specs=[
            pl.BlockSpec((bn, 3), lambda i: (i, 0)),
            pl.BlockSpec((_S, 3), lambda i: (0, 0)),
        ],
        out_specs=[pl.BlockSpec((bn, _S), lambda i: (i, 0))] * 3,
        out_shape=out_shape,
    )(a, s)

    return jnp.stack([stone_s, dist_s, ang_s], axis=-1)


# speedup vs baseline: 94.0732x; 94.0732x over previous
import functools
import math

import jax
import jax.numpy as jnp
from jax import lax
from jax.experimental import pallas as pl
from jax.experimental.pallas import tpu as pltpu
from jax.experimental.pallas import tpu_sc as plsc

_S = 256
_L = 16
_NW = 32

_HALF_PI = math.pi / 2.0
_PI = math.pi
_R2D = 180.0 / math.pi

_ATAN_C = (0.9999965494666837, -0.33318339140806397, 0.19814843475101232,
           -0.1325642608068135, 0.0800028446589445, -0.033907658670957394,
           0.006905941419869666)


def _rsqrt(x):
    i = plsc.bitcast(x, jnp.int32)
    i = 0x5F3759DF - (i >> 1)
    y = plsc.bitcast(i, jnp.float32)
    for _ in range(3):
        y = y * (1.5 - 0.5 * x * y * y)
    return y


def _atan2deg(n, d):
    an = jnp.abs(n)
    ad = jnp.abs(d)
    mx = jnp.maximum(an, ad)
    mn = jnp.minimum(an, ad)
    t = jnp.where(mx > 0, mn / mx, 0.0)
    t2 = t * t
    z = jnp.full((_L,), _ATAN_C[-1], jnp.float32)
    for c in reversed(_ATAN_C[:-1]):
        z = z * t2 + c
    z = z * t
    z = jnp.where(an > ad, _HALF_PI - z, z)
    z = jnp.where(d < 0, _PI - z, z)
    z = jnp.where(n < 0, -z, z)
    deg = z * _R2D
    return jnp.where(deg > 0, deg, deg + 360.0)


def _sc_sort_call(occ, ay, ax, sval, sy, sx, interpret=False):
    n = occ.shape[0]
    rows_w = n // _NW
    groups = rows_w // _L
    mesh = plsc.VectorSubcoreMesh(core_axis_name="c", subcore_axis_name="s")

    @functools.partial(
        pl.kernel,
        out_type=jax.ShapeDtypeStruct((n, 3 * _S), jnp.float32),
        mesh=mesh,
        scratch_types=[
            pltpu.VMEM((_S,), jnp.float32),
            pltpu.VMEM((_S,), jnp.float32),
            pltpu.VMEM((_S,), jnp.float32),
            pltpu.VMEM((_L,), jnp.float32),
            pltpu.VMEM((_L,), jnp.float32),
            pltpu.VMEM((_L,), jnp.float32),
            pltpu.VMEM((_S, _L), jnp.float32),
            pltpu.VMEM((_S, _L), jnp.int32),
            pltpu.VMEM((_L, 3 * _S), jnp.float32),
        ],
        interpret=interpret,
        compiler_params=pltpu.CompilerParams(needs_layout_passes=False),
    )
    def k(occ_h, ay_h, ax_h, sval_h, sy_h, sx_h, out_h,
          sval_v, sy_v, sx_v, occ_v, ay_v, ax_v, d2_t, idx_t, out_v):
        wid = lax.axis_index("s") * 2 + lax.axis_index("c")
        pltpu.sync_copy(sval_h, sval_v)
        pltpu.sync_copy(sy_h, sy_v)
        pltpu.sync_copy(sx_h, sx_v)
        lanev = lax.iota(jnp.int32, _L)

        def group_body(g, _):
            base = wid * rows_w + g * _L
            pltpu.sync_copy(occ_h.at[pl.ds(base, _L)], occ_v)
            pltpu.sync_copy(ay_h.at[pl.ds(base, _L)], ay_v)
            pltpu.sync_copy(ax_h.at[pl.ds(base, _L)], ax_v)
            ayv = ay_v[...]
            axv = ax_v[...]
            occv = occ_v[...]

            def dstone(ki, _):
                kvec = jnp.full((_L,), ki, jnp.int32)
                syk = plsc.load_gather(sy_v, [kvec])
                sxk = plsc.load_gather(sx_v, [kvec])
                dy = syk - ayv
                dx = sxk - axv
                d2_t[ki] = dy * dy + dx * dx
                idx_t[ki] = kvec
                return 0

            lax.fori_loop(0, _S, dstone, 0, unroll=4)

            for lk in range(1, 9):
                kk = 1 << lk

                def stage(t, _, kk=kk):
                    j = kk >> (1 + t)

                    def ce(i, _):
                        a = 2 * i - (i & (j - 1))
                        b = a + j
                        da = d2_t[a]
                        db = d2_t[b]
                        ia = idx_t[a]
                        ib = idx_t[b]
                        asc = ((a & kk) == 0).astype(jnp.int32)
                        lt = ((da < db) | ((da == db) & (ia < ib)))
                        sel = lt.astype(jnp.int32) == jnp.full((_L,), asc)
                        d2_t[a] = jnp.where(sel, da, db)
                        d2_t[b] = jnp.where(sel, db, da)
                        idx_t[a] = jnp.where(sel, ia, ib)
                        idx_t[b] = jnp.where(sel, ib, ia)
                        return 0

                    lax.fori_loop(0, _S // 2, ce, 0, unroll=4)
                    return 0

                lax.fori_loop(0, lk, stage, 0)

            maskv = jnp.where(occv == 0.0, 1.0, 0.0)

            def outk(ki, _):
                sidx = idx_t[ki]
                d2s = d2_t[ki]
                sv = plsc.load_gather(sval_v, [sidx])
                syk = plsc.load_gather(sy_v, [sidx])
                sxk = plsc.load_gather(sx_v, [sidx])
                dy = syk - ayv
                dx = sxk - axv
                dist = jnp.where(d2s > 0, d2s * _rsqrt(d2s), 0.0)
                ang = _atan2deg(-dy, dx)
                col = ki * 3
                plsc.store_scatter(out_v, [lanev, jnp.full((_L,), col)],
                                   sv * maskv)
                plsc.store_scatter(out_v, [lanev, jnp.full((_L,), col + 1)],
                                   dist * maskv)
                plsc.store_scatter(out_v, [lanev, jnp.full((_L,), col + 2)],
                                   ang * maskv)
                return 0

            lax.fori_loop(0, _S, outk, 0, unroll=2)
            pltpu.sync_copy(out_v, out_h.at[pl.ds(base, _L)])
            return 0

        lax.fori_loop(0, groups, group_body, 0)

    return k(occ, ay, ax, sval, sy, sx)


@jax.jit
def kernel(all_coord_input, stone_coord_input):
    a = all_coord_input.astype(jnp.float32)
    s = stone_coord_input.astype(jnp.float32)
    n = a.shape[0]
    out = _sc_sort_call(a[:, 0], a[:, 1], a[:, 2],
                        s[:, 0], s[:, 1], s[:, 2])
    return out.reshape(n, _S, 3)
